# baseline (speedup 1.0000x reference)
import jax
import jax.numpy as jnp
from jax.experimental import pallas as pl
from jax.experimental.pallas import tpu as pltpu

N_AGENTS = 32
INPUT_DIM = 4
HIDDEN = 128
HEADS = 4
B = 8
T = 20
PRED_LEN = 12
C_OUT = 64
D_FF = 256
KNN = 8
F32 = jnp.float32


def _ln(h, g, b, eps=1e-5):
    m = h.mean(-1, keepdims=True)
    v = ((h - m) ** 2).mean(-1, keepdims=True)
    return (h - m) / jnp.sqrt(v + eps) * g + b


def _gat_kernel(x_ref,
                wl0, bl0, wr0, br0, att0, bs0, g0, b0,
                wl1, bl1, wr1, br1, att1, bs1, g1, b1,
                out_ref):
    x = x_ref[0]

    pos = x[:, :2]
    diff = pos[:, None, :] - pos[None, :, :]
    d2 = (diff * diff).sum(-1)
    dist = jnp.sqrt(d2)
    sigma = jnp.mean(dist)
    adj = jnp.exp(-(dist * dist) / (2.0 * sigma * sigma))
    ge = (adj[:, None, :] >= adj[:, :, None]).astype(F32)
    r = ge.sum(-1)
    cand = jnp.where(r >= float(KNN), adj, -1.0)
    thr = cand.max(axis=1, keepdims=True)
    mask = adj >= thr

    def gat_layer(h, wl_r, bl_r, wr_r, br_r, att_r, bs_r):
        wl = wl_r[...]
        wr = wr_r[...]
        att = att_r[...]
        hl = jnp.dot(h, wl, preferred_element_type=F32) + bl_r[...]
        hr = jnp.dot(h, wr, preferred_element_type=F32) + br_r[...]
        acc = jnp.zeros((N_AGENTS, HIDDEN), F32)
        for hh in range(HEADS):
            hl_h = hl[:, hh * HIDDEN:(hh + 1) * HIDDEN]
            hr_h = hr[:, hh * HIDDEN:(hh + 1) * HIDDEN]
            s = hl_h[:, None, :] + hr_h[None, :, :]
            s = jnp.where(s >= 0, s, 0.2 * s)
            e = (s * att[hh:hh + 1][None]).sum(-1)
            e = jnp.where(mask, e, -1e9)
            m = e.max(axis=0, keepdims=True)
            a = jnp.exp(e - m)
            alpha = a / a.sum(axis=0, keepdims=True)
            out_h = jax.lax.dot_general(
                alpha, hl_h, (((0,), (0,)), ((), ())),
                preferred_element_type=F32)
            acc = acc + out_h
        return acc * (1.0 / HEADS) + bs_r[...]

    h1 = gat_layer(x, wl0, bl0, wr0, br0, att0, bs0)
    h1 = jnp.maximum(_ln(h1, g0[...], b0[...]), 0.0)
    h2 = gat_layer(h1, wl1, bl1, wr1, br1, att1, bs1)
    h2 = jnp.maximum(_ln(h2, g1[...], b1[...]), 0.0)
    out_ref[0] = h2 + h1


def _gru_dec_kernel(x_ref, pg, pb,
                    wih0, bih0, whh0, bhh0,
                    wih1, bih1, whh1, bhh1,
                    w1, b1, lg, lb, w2, b2,
                    out_ref, s_gi, s_h, s_gi2):
    x = x_ref[...]
    xn = _ln(x, pg[...], pb[...])
    s_gi[...] = jax.lax.dot_general(
        xn, wih0[...], (((1,), (1,)), ((), ())),
        preferred_element_type=F32) + bih0[...]

    whh0v = whh0[...]
    bhh0v = bhh0[...]

    def step0(t, h):
        gi = s_gi[pl.ds(t * B, B), :]
        gh = jax.lax.dot_general(h, whh0v, (((1,), (1,)), ((), ())),
                                 preferred_element_type=F32) + bhh0v
        r = jax.nn.sigmoid(gi[:, :HIDDEN] + gh[:, :HIDDEN])
        z = jax.nn.sigmoid(gi[:, HIDDEN:2 * HIDDEN] + gh[:, HIDDEN:2 * HIDDEN])
        n = jnp.tanh(gi[:, 2 * HIDDEN:] + r * gh[:, 2 * HIDDEN:])
        hn = (1.0 - z) * n + z * h
        s_h[pl.ds(t * B, B), :] = hn
        return hn

    jax.lax.fori_loop(0, T, step0, jnp.zeros((B, HIDDEN), F32))

    s_gi2[...] = jax.lax.dot_general(
        s_h[...], wih1[...], (((1,), (1,)), ((), ())),
        preferred_element_type=F32) + bih1[...]

    whh1v = whh1[...]
    bhh1v = bhh1[...]

    def step1(t, h):
        gi = s_gi2[pl.ds(t * B, B), :]
        gh = jax.lax.dot_general(h, whh1v, (((1,), (1,)), ((), ())),
                                 preferred_element_type=F32) + bhh1v
        r = jax.nn.sigmoid(gi[:, :HIDDEN] + gh[:, :HIDDEN])
        z = jax.nn.sigmoid(gi[:, HIDDEN:2 * HIDDEN] + gh[:, HIDDEN:2 * HIDDEN])
        n = jnp.tanh(gi[:, 2 * HIDDEN:] + r * gh[:, 2 * HIDDEN:])
        return (1.0 - z) * n + z * h

    h = jax.lax.fori_loop(0, T, step1, jnp.zeros((B, HIDDEN), F32))

    d = jnp.dot(h, w1[...], preferred_element_type=F32) + b1[...]
    d = jnp.maximum(_ln(d, lg[...], lb[...]), 0.0)
    out_ref[...] = jnp.dot(d, w2[...], preferred_element_type=F32) + b2[...]


def _row2(a):
    return a.reshape(1, -1)


def kernel(x_enc, x_mark_enc, x_dec, x_mark_dec, params):
    b, t = x_enc.shape[0], x_enc.shape[1]
    x = x_enc[:, :, -N_AGENTS * INPUT_DIM:].reshape(b * t, N_AGENTS, INPUT_DIM)

    gat_ws = []
    gat_specs = []
    for i in range(E := 2):
        p = params['gat'][i]
        ln = params['gat_ln'][i]
        ops = [p['lin_l_w'], _row2(p['lin_l_b']), p['lin_r_w'],
               _row2(p['lin_r_b']), p['att'], _row2(p['bias']),
               _row2(ln['g']), _row2(ln['b'])]
        gat_ws += ops
        gat_specs += [pl.BlockSpec(o.shape, lambda i, n=o.ndim: (0,) * n)
                      for o in ops]

    gt = b * t
    out1 = pl.pallas_call(
        _gat_kernel,
        grid=(gt,),
        in_specs=[pl.BlockSpec((1, N_AGENTS, INPUT_DIM), lambda i: (i, 0, 0))]
        + gat_specs,
        out_specs=pl.BlockSpec((1, N_AGENTS, HIDDEN), lambda i: (i, 0, 0)),
        out_shape=jax.ShapeDtypeStruct((gt, N_AGENTS, HIDDEN), F32),
    )(x, *gat_ws)

    gat_flat = out1.reshape(b, t, N_AGENTS * HIDDEN).transpose(1, 0, 2)
    gat_flat = gat_flat.reshape(b * t, N_AGENTS * HIDDEN)

    g0, g1 = params['gru']
    d = params['dec']
    ops2 = [gat_flat, _row2(params['pre_ln']['g']), _row2(params['pre_ln']['b']),
            g0['w_ih'], _row2(g0['b_ih']), g0['w_hh'], _row2(g0['b_hh']),
            g1['w_ih'], _row2(g1['b_ih']), g1['w_hh'], _row2(g1['b_hh']),
            d['w1'], _row2(d['b1']), _row2(d['ln_g']), _row2(d['ln_b']),
            d['w2'], _row2(d['b2'])]

    out2 = pl.pallas_call(
        _gru_dec_kernel,
        out_shape=jax.ShapeDtypeStruct((B, PRED_LEN * C_OUT), F32),
        scratch_shapes=[pltpu.VMEM((gt, 3 * HIDDEN), F32),
                        pltpu.VMEM((gt, HIDDEN), F32),
                        pltpu.VMEM((gt, 3 * HIDDEN), F32)],
    )(*ops2)

    return out2.reshape(b, PRED_LEN, C_OUT)


# speedup vs baseline: 1.9500x; 1.9500x over previous
import jax
import jax.numpy as jnp
from jax.experimental import pallas as pl
from jax.experimental.pallas import tpu as pltpu

N_AGENTS = 32
INPUT_DIM = 4
HIDDEN = 128
HEADS = 4
B = 8
T = 20
PRED_LEN = 12
C_OUT = 64
D_FF = 256
KNN = 8
F32 = jnp.float32


def _ln(h, g, b, eps=1e-5):
    m = h.mean(-1, keepdims=True)
    v = ((h - m) ** 2).mean(-1, keepdims=True)
    return (h - m) / jnp.sqrt(v + eps) * g + b


def _gat_kernel(x_ref,
                wl0, bl0, wr0, br0, att0, bs0, g0, b0,
                wl1, bl1, wr1, br1, att1, bs1, g1, b1,
                out_ref):
    x = x_ref[0]

    pos = x[:, :2]
    diff = pos[:, None, :] - pos[None, :, :]
    d2 = (diff * diff).sum(-1)
    lane = jax.lax.broadcasted_iota(jnp.int32, (N_AGENTS, N_AGENTS), 1)
    work = d2
    for k in range(KNN - 1):
        cur = work.min(axis=1, keepdims=True)
        ismin = jnp.where(work <= cur, lane, N_AGENTS)
        first = ismin.min(axis=1, keepdims=True)
        work = jnp.where(lane == first, jnp.inf, work)
    thr = work.min(axis=1, keepdims=True)
    mask = d2 <= thr

    ones_agg = jnp.ones((N_AGENTS, HIDDEN), F32)

    def gat_layer(h, wl_r, bl_r, wr_r, br_r, att_r, bs_r):
        wl = wl_r[...]
        wr = wr_r[...]
        att = att_r[...]
        hl = jnp.dot(h, wl, preferred_element_type=F32) + bl_r[...]
        hr = jnp.dot(h, wr, preferred_element_type=F32) + br_r[...]
        acc = jnp.zeros((N_AGENTS, HIDDEN), F32)
        for hh in range(HEADS):
            hl_h = hl[:, hh * HIDDEN:(hh + 1) * HIDDEN]
            hr_h = hr[:, hh * HIDDEN:(hh + 1) * HIDDEN]
            s = hl_h[:, None, :] + hr_h[None, :, :]
            s = jnp.maximum(s, 0.2 * s)
            e = (s * att[hh:hh + 1][None]).sum(-1)
            a = jnp.where(mask, jnp.exp(e), 0.0)
            num = jax.lax.dot_general(
                a, hl_h, (((0,), (0,)), ((), ())),
                preferred_element_type=F32)
            den = jax.lax.dot_general(
                a, ones_agg, (((0,), (0,)), ((), ())),
                preferred_element_type=F32)
            acc = acc + num / den
        return acc * (1.0 / HEADS) + bs_r[...]

    h1 = gat_layer(x, wl0, bl0, wr0, br0, att0, bs0)
    h1 = jnp.maximum(_ln(h1, g0[...], b0[...]), 0.0)
    h2 = gat_layer(h1, wl1, bl1, wr1, br1, att1, bs1)
    h2 = jnp.maximum(_ln(h2, g1[...], b1[...]), 0.0)
    out_ref[0] = h2 + h1


def _gru_dec_kernel(x_ref, pg, pb,
                    wih0, bih0, whh0, bhh0,
                    wih1, bih1, whh1, bhh1,
                    w1, b1, lg, lb, w2, b2,
                    out_ref, s_gi, s_h, s_gi2):
    x = x_ref[...]
    xn = _ln(x, pg[...], pb[...])
    s_gi[...] = jax.lax.dot_general(
        xn, wih0[...], (((1,), (1,)), ((), ())),
        preferred_element_type=F32) + bih0[...]

    whh0v = whh0[...]
    bhh0v = bhh0[...]

    def step0(t, h):
        gi = s_gi[pl.ds(t * B, B), :]
        gh = jax.lax.dot_general(h, whh0v, (((1,), (1,)), ((), ())),
                                 preferred_element_type=F32) + bhh0v
        r = jax.nn.sigmoid(gi[:, :HIDDEN] + gh[:, :HIDDEN])
        z = jax.nn.sigmoid(gi[:, HIDDEN:2 * HIDDEN] + gh[:, HIDDEN:2 * HIDDEN])
        n = jnp.tanh(gi[:, 2 * HIDDEN:] + r * gh[:, 2 * HIDDEN:])
        hn = (1.0 - z) * n + z * h
        s_h[pl.ds(t * B, B), :] = hn
        return hn

    jax.lax.fori_loop(0, T, step0, jnp.zeros((B, HIDDEN), F32))

    s_gi2[...] = jax.lax.dot_general(
        s_h[...], wih1[...], (((1,), (1,)), ((), ())),
        preferred_element_type=F32) + bih1[...]

    whh1v = whh1[...]
    bhh1v = bhh1[...]

    def step1(t, h):
        gi = s_gi2[pl.ds(t * B, B), :]
        gh = jax.lax.dot_general(h, whh1v, (((1,), (1,)), ((), ())),
                                 preferred_element_type=F32) + bhh1v
        r = jax.nn.sigmoid(gi[:, :HIDDEN] + gh[:, :HIDDEN])
        z = jax.nn.sigmoid(gi[:, HIDDEN:2 * HIDDEN] + gh[:, HIDDEN:2 * HIDDEN])
        n = jnp.tanh(gi[:, 2 * HIDDEN:] + r * gh[:, 2 * HIDDEN:])
        return (1.0 - z) * n + z * h

    h = jax.lax.fori_loop(0, T, step1, jnp.zeros((B, HIDDEN), F32))

    d = jnp.dot(h, w1[...], preferred_element_type=F32) + b1[...]
    d = jnp.maximum(_ln(d, lg[...], lb[...]), 0.0)
    out_ref[...] = jnp.dot(d, w2[...], preferred_element_type=F32) + b2[...]


def _row2(a):
    return a.reshape(1, -1)


def kernel(x_enc, x_mark_enc, x_dec, x_mark_dec, params):
    b, t = x_enc.shape[0], x_enc.shape[1]
    x = x_enc[:, :, -N_AGENTS * INPUT_DIM:].reshape(b * t, N_AGENTS, INPUT_DIM)

    gat_ws = []
    gat_specs = []
    for i in range(E := 2):
        p = params['gat'][i]
        ln = params['gat_ln'][i]
        ops = [p['lin_l_w'], _row2(p['lin_l_b']), p['lin_r_w'],
               _row2(p['lin_r_b']), p['att'], _row2(p['bias']),
               _row2(ln['g']), _row2(ln['b'])]
        gat_ws += ops
        gat_specs += [pl.BlockSpec(o.shape, lambda i, n=o.ndim: (0,) * n)
                      for o in ops]

    gt = b * t
    out1 = pl.pallas_call(
        _gat_kernel,
        grid=(gt,),
        in_specs=[pl.BlockSpec((1, N_AGENTS, INPUT_DIM), lambda i: (i, 0, 0))]
        + gat_specs,
        out_specs=pl.BlockSpec((1, N_AGENTS, HIDDEN), lambda i: (i, 0, 0)),
        out_shape=jax.ShapeDtypeStruct((gt, N_AGENTS, HIDDEN), F32),
    )(x, *gat_ws)

    gat_flat = out1.reshape(b, t, N_AGENTS * HIDDEN).transpose(1, 0, 2)
    gat_flat = gat_flat.reshape(b * t, N_AGENTS * HIDDEN)

    g0, g1 = params['gru']
    d = params['dec']
    ops2 = [gat_flat, _row2(params['pre_ln']['g']), _row2(params['pre_ln']['b']),
            g0['w_ih'], _row2(g0['b_ih']), g0['w_hh'], _row2(g0['b_hh']),
            g1['w_ih'], _row2(g1['b_ih']), g1['w_hh'], _row2(g1['b_hh']),
            d['w1'], _row2(d['b1']), _row2(d['ln_g']), _row2(d['ln_b']),
            d['w2'], _row2(d['b2'])]

    out2 = pl.pallas_call(
        _gru_dec_kernel,
        out_shape=jax.ShapeDtypeStruct((B, PRED_LEN * C_OUT), F32),
        scratch_shapes=[pltpu.VMEM((gt, 3 * HIDDEN), F32),
                        pltpu.VMEM((gt, HIDDEN), F32),
                        pltpu.VMEM((gt, 3 * HIDDEN), F32)],
    )(*ops2)

    return out2.reshape(b, PRED_LEN, C_OUT)


# speedup vs baseline: 2.3282x; 1.1940x over previous
import jax
import jax.numpy as jnp
from jax.experimental import pallas as pl
from jax.experimental.pallas import tpu as pltpu

N_AGENTS = 32
INPUT_DIM = 4
HIDDEN = 128
HEADS = 4
B = 8
T = 20
PRED_LEN = 12
C_OUT = 64
D_FF = 256
KNN = 8
F32 = jnp.float32


def _ln(h, g, b, eps=1e-5):
    m = h.mean(-1, keepdims=True)
    v = ((h - m) ** 2).mean(-1, keepdims=True)
    return (h - m) / jnp.sqrt(v + eps) * g + b


def _gat_kernel(x_ref,
                wl0, bl0, wr0, br0, att0, bs0, g0, b0,
                wl1, bl1, wr1, br1, att1, bs1, g1, b1,
                out_ref):
    x = x_ref[0]

    pos = x[:, :2]
    diff = pos[:, None, :] - pos[None, :, :]
    d2 = (diff * diff).sum(-1)
    lane = jax.lax.broadcasted_iota(jnp.int32, (N_AGENTS, N_AGENTS), 1)
    work = d2
    for k in range(KNN - 1):
        cur = work.min(axis=1, keepdims=True)
        ismin = jnp.where(work <= cur, lane, N_AGENTS)
        first = ismin.min(axis=1, keepdims=True)
        work = jnp.where(lane == first, jnp.inf, work)
    thr = work.min(axis=1, keepdims=True)
    mask = d2 <= thr

    maskf = mask.astype(F32)
    mask4 = jnp.concatenate([maskf] * HEADS, axis=1)

    def gat_layer(h, wl_r, bl_r, wr_r, br_r, att_r, bs_r):
        wl = wl_r[...]
        wr = wr_r[...]
        att = att_r[...]
        hl = jnp.dot(h, wl, preferred_element_type=F32) + bl_r[...]
        hr = jnp.dot(h, wr, preferred_element_type=F32) + br_r[...]
        e_cols = []
        for hh in range(HEADS):
            hl_h = hl[:, hh * HIDDEN:(hh + 1) * HIDDEN]
            hr_h = hr[:, hh * HIDDEN:(hh + 1) * HIDDEN]
            att_h = att[hh:hh + 1]
            chunks = []
            for ib in range(4):
                s = hl_h[ib * 8:(ib + 1) * 8][:, None, :] + hr_h[None, :, :]
                s = jnp.maximum(s, 0.2 * s)
                chunks.append((s * att_h[None]).sum(-1))
            e_cols.append(jnp.concatenate(chunks, axis=0))
        e_all = jnp.concatenate(e_cols, axis=1)
        a_all = jnp.exp(e_all) * mask4
        den = a_all.sum(axis=0, keepdims=True)
        alpha = a_all / den
        acc = jnp.zeros((N_AGENTS, HIDDEN), F32)
        for hh in range(HEADS):
            hl_h = hl[:, hh * HIDDEN:(hh + 1) * HIDDEN]
            acc = acc + jax.lax.dot_general(
                alpha[:, hh * N_AGENTS:(hh + 1) * N_AGENTS], hl_h,
                (((0,), (0,)), ((), ())),
                preferred_element_type=F32)
        return acc * (1.0 / HEADS) + bs_r[...]

    h1 = gat_layer(x, wl0, bl0, wr0, br0, att0, bs0)
    h1 = jnp.maximum(_ln(h1, g0[...], b0[...]), 0.0)
    h2 = gat_layer(h1, wl1, bl1, wr1, br1, att1, bs1)
    h2 = jnp.maximum(_ln(h2, g1[...], b1[...]), 0.0)
    out_ref[0] = h2 + h1


def _gru_dec_kernel(x_ref, pg, pb,
                    wih0, bih0, whh0, bhh0,
                    wih1, bih1, whh1, bhh1,
                    w1, b1, lg, lb, w2, b2,
                    out_ref, s_gi, s_h, s_gi2):
    x = x_ref[...]
    xn = _ln(x, pg[...], pb[...])
    s_gi[...] = jax.lax.dot_general(
        xn, wih0[...], (((1,), (1,)), ((), ())),
        preferred_element_type=F32) + bih0[...]

    whh0v = whh0[...]
    bhh0v = bhh0[...]

    def step0(t, h):
        gi = s_gi[pl.ds(t * B, B), :]
        gh = jax.lax.dot_general(h, whh0v, (((1,), (1,)), ((), ())),
                                 preferred_element_type=F32) + bhh0v
        r = jax.nn.sigmoid(gi[:, :HIDDEN] + gh[:, :HIDDEN])
        z = jax.nn.sigmoid(gi[:, HIDDEN:2 * HIDDEN] + gh[:, HIDDEN:2 * HIDDEN])
        n = jnp.tanh(gi[:, 2 * HIDDEN:] + r * gh[:, 2 * HIDDEN:])
        hn = (1.0 - z) * n + z * h
        s_h[pl.ds(t * B, B), :] = hn
        return hn

    jax.lax.fori_loop(0, T, step0, jnp.zeros((B, HIDDEN), F32))

    s_gi2[...] = jax.lax.dot_general(
        s_h[...], wih1[...], (((1,), (1,)), ((), ())),
        preferred_element_type=F32) + bih1[...]

    whh1v = whh1[...]
    bhh1v = bhh1[...]

    def step1(t, h):
        gi = s_gi2[pl.ds(t * B, B), :]
        gh = jax.lax.dot_general(h, whh1v, (((1,), (1,)), ((), ())),
                                 preferred_element_type=F32) + bhh1v
        r = jax.nn.sigmoid(gi[:, :HIDDEN] + gh[:, :HIDDEN])
        z = jax.nn.sigmoid(gi[:, HIDDEN:2 * HIDDEN] + gh[:, HIDDEN:2 * HIDDEN])
        n = jnp.tanh(gi[:, 2 * HIDDEN:] + r * gh[:, 2 * HIDDEN:])
        return (1.0 - z) * n + z * h

    h = jax.lax.fori_loop(0, T, step1, jnp.zeros((B, HIDDEN), F32))

    d = jnp.dot(h, w1[...], preferred_element_type=F32) + b1[...]
    d = jnp.maximum(_ln(d, lg[...], lb[...]), 0.0)
    out_ref[...] = jnp.dot(d, w2[...], preferred_element_type=F32) + b2[...]


def _row2(a):
    return a.reshape(1, -1)


def kernel(x_enc, x_mark_enc, x_dec, x_mark_dec, params):
    b, t = x_enc.shape[0], x_enc.shape[1]
    x = x_enc[:, :, -N_AGENTS * INPUT_DIM:].reshape(b * t, N_AGENTS, INPUT_DIM)

    gat_ws = []
    gat_specs = []
    for i in range(E := 2):
        p = params['gat'][i]
        ln = params['gat_ln'][i]
        ops = [p['lin_l_w'], _row2(p['lin_l_b']), p['lin_r_w'],
               _row2(p['lin_r_b']), p['att'], _row2(p['bias']),
               _row2(ln['g']), _row2(ln['b'])]
        gat_ws += ops
        gat_specs += [pl.BlockSpec(o.shape, lambda i, n=o.ndim: (0,) * n)
                      for o in ops]

    gt = b * t
    out1 = pl.pallas_call(
        _gat_kernel,
        grid=(gt,),
        in_specs=[pl.BlockSpec((1, N_AGENTS, INPUT_DIM), lambda i: (i, 0, 0))]
        + gat_specs,
        out_specs=pl.BlockSpec((1, N_AGENTS, HIDDEN), lambda i: (i, 0, 0)),
        out_shape=jax.ShapeDtypeStruct((gt, N_AGENTS, HIDDEN), F32),
    )(x, *gat_ws)

    gat_flat = out1.reshape(b, t, N_AGENTS * HIDDEN).transpose(1, 0, 2)
    gat_flat = gat_flat.reshape(b * t, N_AGENTS * HIDDEN)

    g0, g1 = params['gru']
    d = params['dec']
    ops2 = [gat_flat, _row2(params['pre_ln']['g']), _row2(params['pre_ln']['b']),
            g0['w_ih'], _row2(g0['b_ih']), g0['w_hh'], _row2(g0['b_hh']),
            g1['w_ih'], _row2(g1['b_ih']), g1['w_hh'], _row2(g1['b_hh']),
            d['w1'], _row2(d['b1']), _row2(d['ln_g']), _row2(d['ln_b']),
            d['w2'], _row2(d['b2'])]

    out2 = pl.pallas_call(
        _gru_dec_kernel,
        out_shape=jax.ShapeDtypeStruct((B, PRED_LEN * C_OUT), F32),
        scratch_shapes=[pltpu.VMEM((gt, 3 * HIDDEN), F32),
                        pltpu.VMEM((gt, HIDDEN), F32),
                        pltpu.VMEM((gt, 3 * HIDDEN), F32)],
    )(*ops2)

    return out2.reshape(b, PRED_LEN, C_OUT)


# speedup vs baseline: 2.4104x; 1.0353x over previous
import jax
import jax.numpy as jnp
from jax.experimental import pallas as pl
from jax.experimental.pallas import tpu as pltpu

N_AGENTS = 32
INPUT_DIM = 4
HIDDEN = 128
HEADS = 4
B = 8
T = 20
PRED_LEN = 12
C_OUT = 64
D_FF = 256
KNN = 8
F32 = jnp.float32


def _ln(h, g, b, eps=1e-5):
    m = h.mean(-1, keepdims=True)
    v = ((h - m) ** 2).mean(-1, keepdims=True)
    return (h - m) / jnp.sqrt(v + eps) * g + b


GPB = 4


def _gat_kernel(x_ref,
                wl0, bl0, wr0, br0, att0, bs0, g0, b0,
                wl1, bl1, wr1, br1, att1, bs1, g1, b1,
                out_ref):
    for g in range(GPB):
        _gat_one_graph(x_ref[g], wl0, bl0, wr0, br0, att0, bs0, g0, b0,
                       wl1, bl1, wr1, br1, att1, bs1, g1, b1, out_ref, g)


def _gat_one_graph(x,
                   wl0, bl0, wr0, br0, att0, bs0, g0, b0,
                   wl1, bl1, wr1, br1, att1, bs1, g1, b1,
                   out_ref, g):

    pos = x[:, :2]
    diff = pos[:, None, :] - pos[None, :, :]
    d2 = (diff * diff).sum(-1)
    lane = jax.lax.broadcasted_iota(jnp.int32, (N_AGENTS, N_AGENTS), 1)
    work = d2
    for k in range(KNN - 1):
        cur = work.min(axis=1, keepdims=True)
        ismin = jnp.where(work <= cur, lane, N_AGENTS)
        first = ismin.min(axis=1, keepdims=True)
        work = jnp.where(lane == first, jnp.inf, work)
    thr = work.min(axis=1, keepdims=True)
    mask = d2 <= thr

    maskf = mask.astype(F32)
    mask4 = jnp.concatenate([maskf] * HEADS, axis=1)

    def gat_layer(h, wl_r, bl_r, wr_r, br_r, att_r, bs_r):
        wl = wl_r[...]
        wr = wr_r[...]
        att = att_r[...]
        hl = jnp.dot(h, wl, preferred_element_type=F32) + bl_r[...]
        hr = jnp.dot(h, wr, preferred_element_type=F32) + br_r[...]
        e_cols = []
        for hh in range(HEADS):
            hl_h = hl[:, hh * HIDDEN:(hh + 1) * HIDDEN]
            hr_h = hr[:, hh * HIDDEN:(hh + 1) * HIDDEN]
            att_h = att[hh:hh + 1]
            chunks = []
            for ib in range(4):
                s = hl_h[ib * 8:(ib + 1) * 8][:, None, :] + hr_h[None, :, :]
                s = jnp.maximum(s, 0.2 * s)
                chunks.append((s * att_h[None]).sum(-1))
            e_cols.append(jnp.concatenate(chunks, axis=0))
        e_all = jnp.concatenate(e_cols, axis=1)
        a_all = jnp.exp(e_all) * mask4
        den = a_all.sum(axis=0, keepdims=True)
        alpha = a_all / den
        acc = jnp.zeros((N_AGENTS, HIDDEN), F32)
        for hh in range(HEADS):
            hl_h = hl[:, hh * HIDDEN:(hh + 1) * HIDDEN]
            acc = acc + jax.lax.dot_general(
                alpha[:, hh * N_AGENTS:(hh + 1) * N_AGENTS], hl_h,
                (((0,), (0,)), ((), ())),
                preferred_element_type=F32)
        return acc * (1.0 / HEADS) + bs_r[...]

    h1 = gat_layer(x, wl0, bl0, wr0, br0, att0, bs0)
    h1 = jnp.maximum(_ln(h1, g0[...], b0[...]), 0.0)
    h2 = gat_layer(h1, wl1, bl1, wr1, br1, att1, bs1)
    h2 = jnp.maximum(_ln(h2, g1[...], b1[...]), 0.0)
    out_ref[g] = h2 + h1


def _gru_dec_kernel(x_ref, pg, pb,
                    wih0, bih0, whh0, bhh0,
                    wih1, bih1, whh1, bhh1,
                    w1, b1, lg, lb, w2, b2,
                    out_ref, s_gi, s_h, s_gi2):
    x = x_ref[...]
    xn = _ln(x, pg[...], pb[...])
    s_gi[...] = jax.lax.dot_general(
        xn, wih0[...], (((1,), (1,)), ((), ())),
        preferred_element_type=F32) + bih0[...]

    whh0v = whh0[...]
    bhh0v = bhh0[...]

    def step0(t, h):
        gi = s_gi[pl.ds(t * B, B), :]
        gh = jax.lax.dot_general(h, whh0v, (((1,), (1,)), ((), ())),
                                 preferred_element_type=F32) + bhh0v
        r = jax.nn.sigmoid(gi[:, :HIDDEN] + gh[:, :HIDDEN])
        z = jax.nn.sigmoid(gi[:, HIDDEN:2 * HIDDEN] + gh[:, HIDDEN:2 * HIDDEN])
        n = jnp.tanh(gi[:, 2 * HIDDEN:] + r * gh[:, 2 * HIDDEN:])
        hn = (1.0 - z) * n + z * h
        s_h[pl.ds(t * B, B), :] = hn
        return hn

    jax.lax.fori_loop(0, T, step0, jnp.zeros((B, HIDDEN), F32))

    s_gi2[...] = jax.lax.dot_general(
        s_h[...], wih1[...], (((1,), (1,)), ((), ())),
        preferred_element_type=F32) + bih1[...]

    whh1v = whh1[...]
    bhh1v = bhh1[...]

    def step1(t, h):
        gi = s_gi2[pl.ds(t * B, B), :]
        gh = jax.lax.dot_general(h, whh1v, (((1,), (1,)), ((), ())),
                                 preferred_element_type=F32) + bhh1v
        r = jax.nn.sigmoid(gi[:, :HIDDEN] + gh[:, :HIDDEN])
        z = jax.nn.sigmoid(gi[:, HIDDEN:2 * HIDDEN] + gh[:, HIDDEN:2 * HIDDEN])
        n = jnp.tanh(gi[:, 2 * HIDDEN:] + r * gh[:, 2 * HIDDEN:])
        return (1.0 - z) * n + z * h

    h = jax.lax.fori_loop(0, T, step1, jnp.zeros((B, HIDDEN), F32))

    d = jnp.dot(h, w1[...], preferred_element_type=F32) + b1[...]
    d = jnp.maximum(_ln(d, lg[...], lb[...]), 0.0)
    out_ref[...] = jnp.dot(d, w2[...], preferred_element_type=F32) + b2[...]


def _row2(a):
    return a.reshape(1, -1)


def kernel(x_enc, x_mark_enc, x_dec, x_mark_dec, params):
    b, t = x_enc.shape[0], x_enc.shape[1]
    x = x_enc[:, :, -N_AGENTS * INPUT_DIM:].reshape(b * t, N_AGENTS, INPUT_DIM)

    gat_ws = []
    gat_specs = []
    for i in range(E := 2):
        p = params['gat'][i]
        ln = params['gat_ln'][i]
        ops = [p['lin_l_w'], _row2(p['lin_l_b']), p['lin_r_w'],
               _row2(p['lin_r_b']), p['att'], _row2(p['bias']),
               _row2(ln['g']), _row2(ln['b'])]
        gat_ws += ops
        gat_specs += [pl.BlockSpec(o.shape, lambda i, n=o.ndim: (0,) * n)
                      for o in ops]

    gt = b * t
    out1 = pl.pallas_call(
        _gat_kernel,
        grid=(gt // GPB,),
        in_specs=[pl.BlockSpec((GPB, N_AGENTS, INPUT_DIM),
                               lambda i: (i, 0, 0))]
        + gat_specs,
        out_specs=pl.BlockSpec((GPB, N_AGENTS, HIDDEN), lambda i: (i, 0, 0)),
        out_shape=jax.ShapeDtypeStruct((gt, N_AGENTS, HIDDEN), F32),
    )(x, *gat_ws)

    gat_flat = out1.reshape(b, t, N_AGENTS * HIDDEN).transpose(1, 0, 2)
    gat_flat = gat_flat.reshape(b * t, N_AGENTS * HIDDEN)

    g0, g1 = params['gru']
    d = params['dec']
    ops2 = [gat_flat, _row2(params['pre_ln']['g']), _row2(params['pre_ln']['b']),
            g0['w_ih'], _row2(g0['b_ih']), g0['w_hh'], _row2(g0['b_hh']),
            g1['w_ih'], _row2(g1['b_ih']), g1['w_hh'], _row2(g1['b_hh']),
            d['w1'], _row2(d['b1']), _row2(d['ln_g']), _row2(d['ln_b']),
            d['w2'], _row2(d['b2'])]

    out2 = pl.pallas_call(
        _gru_dec_kernel,
        out_shape=jax.ShapeDtypeStruct((B, PRED_LEN * C_OUT), F32),
        scratch_shapes=[pltpu.VMEM((gt, 3 * HIDDEN), F32),
                        pltpu.VMEM((gt, HIDDEN), F32),
                        pltpu.VMEM((gt, 3 * HIDDEN), F32)],
    )(*ops2)

    return out2.reshape(b, PRED_LEN, C_OUT)


# speedup vs baseline: 3.1708x; 1.3155x over previous
import jax
import jax.numpy as jnp
from jax.experimental import pallas as pl
from jax.experimental.pallas import tpu as pltpu

N_AGENTS = 32
INPUT_DIM = 4
HIDDEN = 128
HEADS = 4
B = 8
T = 20
PRED_LEN = 12
C_OUT = 64
D_FF = 256
KNN = 8
F32 = jnp.float32


def _ln(h, g, b, eps=1e-5):
    m = h.mean(-1, keepdims=True)
    v = ((h - m) ** 2).mean(-1, keepdims=True)
    return (h - m) / jnp.sqrt(v + eps) * g + b


GPB = 4


def _knn_mask4(x):
    pos = x[:, :2]
    diff = pos[:, None, :] - pos[None, :, :]
    d2 = (diff * diff).sum(-1)
    lane = jax.lax.broadcasted_iota(jnp.int32, (N_AGENTS, N_AGENTS), 1)
    work = d2
    for k in range(KNN - 1):
        cur = work.min(axis=1, keepdims=True)
        ismin = jnp.where(work <= cur, lane, N_AGENTS)
        first = ismin.min(axis=1, keepdims=True)
        work = jnp.where(lane == first, jnp.inf, work)
    thr = work.min(axis=1, keepdims=True)
    maskf = (d2 <= thr).astype(F32)
    return jnp.concatenate([maskf] * HEADS, axis=1)


def _gat_layer_multi(hs, mask4s, wl_r, bl_r, wr_r, br_r, att_r, bs_r):
    wl = wl_r[...]
    wr = wr_r[...]
    att = att_r[...]
    bl = bl_r[...]
    br = br_r[...]
    bs = bs_r[...]
    hls = [jnp.dot(h, wl, preferred_element_type=F32) + bl for h in hs]
    hrs = [jnp.dot(h, wr, preferred_element_type=F32) + br for h in hs]
    chunks = {}
    for hh in range(HEADS):
        att_h = att[hh:hh + 1]
        for ib in range(4):
            for g in range(len(hs)):
                hl_h = hls[g][:, hh * HIDDEN:(hh + 1) * HIDDEN]
                hr_h = hrs[g][:, hh * HIDDEN:(hh + 1) * HIDDEN]
                s = hl_h[ib * 8:(ib + 1) * 8][:, None, :] + hr_h[None, :, :]
                s = jnp.maximum(s, 0.2 * s)
                chunks[(g, hh, ib)] = (s * att_h[None]).sum(-1)
    outs = []
    for g in range(len(hs)):
        e_all = jnp.concatenate(
            [jnp.concatenate([chunks[(g, hh, ib)] for ib in range(4)], axis=0)
             for hh in range(HEADS)], axis=1)
        a_all = jnp.exp(e_all) * mask4s[g]
        den = a_all.sum(axis=0, keepdims=True)
        alpha = a_all / den
        acc = jnp.zeros((N_AGENTS, HIDDEN), F32)
        for hh in range(HEADS):
            hl_h = hls[g][:, hh * HIDDEN:(hh + 1) * HIDDEN]
            acc = acc + jax.lax.dot_general(
                alpha[:, hh * N_AGENTS:(hh + 1) * N_AGENTS], hl_h,
                (((0,), (0,)), ((), ())),
                preferred_element_type=F32)
        outs.append(acc * (1.0 / HEADS) + bs)
    return outs


def _gat_kernel(x_ref,
                wl0, bl0, wr0, br0, att0, bs0, g0, b0,
                wl1, bl1, wr1, br1, att1, bs1, g1, b1,
                out_ref):
    xs = [x_ref[g] for g in range(GPB)]
    mask4s = [_knn_mask4(x) for x in xs]
    h1s = _gat_layer_multi(xs, mask4s, wl0, bl0, wr0, br0, att0, bs0)
    h1s = [jnp.maximum(_ln(h, g0[...], b0[...]), 0.0) for h in h1s]
    h2s = _gat_layer_multi(h1s, mask4s, wl1, bl1, wr1, br1, att1, bs1)
    for g in range(GPB):
        h2 = jnp.maximum(_ln(h2s[g], g1[...], b1[...]), 0.0)
        out_ref[g] = h2 + h1s[g]


def _gru_dec_kernel(x_ref, pg, pb,
                    wih0, bih0, whh0, bhh0,
                    wih1, bih1, whh1, bhh1,
                    w1, b1, lg, lb, w2, b2,
                    out_ref, s_gi, s_h, s_gi2):
    x = x_ref[...]
    xn = _ln(x, pg[...], pb[...])
    s_gi[...] = jax.lax.dot_general(
        xn, wih0[...], (((1,), (1,)), ((), ())),
        preferred_element_type=F32) + bih0[...]

    whh0v = whh0[...]
    bhh0v = bhh0[...]

    def step0(t, h):
        gi = s_gi[pl.ds(t * B, B), :]
        gh = jax.lax.dot_general(h, whh0v, (((1,), (1,)), ((), ())),
                                 preferred_element_type=F32) + bhh0v
        r = jax.nn.sigmoid(gi[:, :HIDDEN] + gh[:, :HIDDEN])
        z = jax.nn.sigmoid(gi[:, HIDDEN:2 * HIDDEN] + gh[:, HIDDEN:2 * HIDDEN])
        n = jnp.tanh(gi[:, 2 * HIDDEN:] + r * gh[:, 2 * HIDDEN:])
        hn = (1.0 - z) * n + z * h
        s_h[pl.ds(t * B, B), :] = hn
        return hn

    jax.lax.fori_loop(0, T, step0, jnp.zeros((B, HIDDEN), F32))

    s_gi2[...] = jax.lax.dot_general(
        s_h[...], wih1[...], (((1,), (1,)), ((), ())),
        preferred_element_type=F32) + bih1[...]

    whh1v = whh1[...]
    bhh1v = bhh1[...]

    def step1(t, h):
        gi = s_gi2[pl.ds(t * B, B), :]
        gh = jax.lax.dot_general(h, whh1v, (((1,), (1,)), ((), ())),
                                 preferred_element_type=F32) + bhh1v
        r = jax.nn.sigmoid(gi[:, :HIDDEN] + gh[:, :HIDDEN])
        z = jax.nn.sigmoid(gi[:, HIDDEN:2 * HIDDEN] + gh[:, HIDDEN:2 * HIDDEN])
        n = jnp.tanh(gi[:, 2 * HIDDEN:] + r * gh[:, 2 * HIDDEN:])
        return (1.0 - z) * n + z * h

    h = jax.lax.fori_loop(0, T, step1, jnp.zeros((B, HIDDEN), F32))

    d = jnp.dot(h, w1[...], preferred_element_type=F32) + b1[...]
    d = jnp.maximum(_ln(d, lg[...], lb[...]), 0.0)
    out_ref[...] = jnp.dot(d, w2[...], preferred_element_type=F32) + b2[...]


def _row2(a):
    return a.reshape(1, -1)


def kernel(x_enc, x_mark_enc, x_dec, x_mark_dec, params):
    b, t = x_enc.shape[0], x_enc.shape[1]
    x = x_enc[:, :, -N_AGENTS * INPUT_DIM:].reshape(b * t, N_AGENTS, INPUT_DIM)

    gat_ws = []
    gat_specs = []
    for i in range(E := 2):
        p = params['gat'][i]
        ln = params['gat_ln'][i]
        ops = [p['lin_l_w'], _row2(p['lin_l_b']), p['lin_r_w'],
               _row2(p['lin_r_b']), p['att'], _row2(p['bias']),
               _row2(ln['g']), _row2(ln['b'])]
        gat_ws += ops
        gat_specs += [pl.BlockSpec(o.shape, lambda i, n=o.ndim: (0,) * n)
                      for o in ops]

    gt = b * t
    out1 = pl.pallas_call(
        _gat_kernel,
        grid=(gt // GPB,),
        in_specs=[pl.BlockSpec((GPB, N_AGENTS, INPUT_DIM),
                               lambda i: (i, 0, 0))]
        + gat_specs,
        out_specs=pl.BlockSpec((GPB, N_AGENTS, HIDDEN), lambda i: (i, 0, 0)),
        out_shape=jax.ShapeDtypeStruct((gt, N_AGENTS, HIDDEN), F32),
    )(x, *gat_ws)

    gat_flat = out1.reshape(b, t, N_AGENTS * HIDDEN).transpose(1, 0, 2)
    gat_flat = gat_flat.reshape(b * t, N_AGENTS * HIDDEN)

    g0, g1 = params['gru']
    d = params['dec']
    ops2 = [gat_flat, _row2(params['pre_ln']['g']), _row2(params['pre_ln']['b']),
            g0['w_ih'], _row2(g0['b_ih']), g0['w_hh'], _row2(g0['b_hh']),
            g1['w_ih'], _row2(g1['b_ih']), g1['w_hh'], _row2(g1['b_hh']),
            d['w1'], _row2(d['b1']), _row2(d['ln_g']), _row2(d['ln_b']),
            d['w2'], _row2(d['b2'])]

    out2 = pl.pallas_call(
        _gru_dec_kernel,
        out_shape=jax.ShapeDtypeStruct((B, PRED_LEN * C_OUT), F32),
        scratch_shapes=[pltpu.VMEM((gt, 3 * HIDDEN), F32),
                        pltpu.VMEM((gt, HIDDEN), F32),
                        pltpu.VMEM((gt, 3 * HIDDEN), F32)],
    )(*ops2)

    return out2.reshape(b, PRED_LEN, C_OUT)
